# baseline (speedup 1.0000x reference)
import functools

import jax
import jax.numpy as jnp
from jax import lax
from jax.experimental import pallas as pl
from jax.experimental.pallas import tpu as pltpu
from jax.experimental.pallas import tpu_sc as plsc

N = 10000
T = 6
M = 10
R = 3
E = 640000
GLI = 14
GLO = 16

DP = 16
NTILES = 32
CH = 256
K = 80
NBUF = 8
EP = NTILES * K * CH
ROWS_PER_TILE = 632
NP = 16 * ROWS_PER_TILE

def _segment_sum_body(r_hbm, src_hbm, dst_hbm, zeros_hbm, out_hbm,
                      src_v, dst_v, rows_v, accum_sh, r_sh,
                      zsem, rsem, gsem, ssem):
    c = lax.axis_index("c")
    s = lax.axis_index("s")
    wid = s * 2 + c
    band = pl.ds(s * ROWS_PER_TILE, ROWS_PER_TILE)

    r_cp = pltpu.async_copy(r_hbm.at[band], r_sh.at[band], rsem)
    zero_cp = pltpu.async_copy(zeros_hbm.at[band], accum_sh.at[band], zsem)
    pltpu.sync_copy(src_hbm.at[wid], src_v)
    pltpu.sync_copy(dst_hbm.at[wid], dst_v)
    r_cp.wait()
    zero_cp.wait()
    plsc.subcore_barrier()

    for b in range(NBUF):
        pltpu.async_copy(r_sh.at[src_v.at[b]], rows_v.at[b], gsem.at[b])

    def group(g, carry):
        for b in range(NBUF):
            j = g * NBUF + b
            pltpu.make_async_copy(r_sh.at[src_v.at[j]], rows_v.at[b],
                                  gsem.at[b]).wait()
            pltpu.async_copy(rows_v.at[b], accum_sh.at[dst_v.at[j]],
                             ssem.at[b], add=True).wait()

            @pl.when(j + NBUF < K)
            def _():
                pltpu.async_copy(r_sh.at[src_v.at[j + NBUF]], rows_v.at[b],
                                 gsem.at[b])
        return carry

    lax.fori_loop(0, K // NBUF, group, 0)

    plsc.subcore_barrier()
    pltpu.sync_copy(accum_sh.at[band], out_hbm.at[c, band])


@functools.cache
def _segment_sum_sc():
    mesh = plsc.VectorSubcoreMesh(core_axis_name="c", subcore_axis_name="s")
    return pl.kernel(
        _segment_sum_body,
        out_type=jax.ShapeDtypeStruct((2, NP, DP), jnp.float32),
        mesh=mesh,
        compiler_params=pltpu.CompilerParams(use_tc_tiling_on_sc=False),
        scratch_types=[
            pltpu.VMEM((K, CH), jnp.int32),
            pltpu.VMEM((K, CH), jnp.int32),
            pltpu.VMEM((NBUF, CH, DP), jnp.float32),
            pltpu.VMEM_SHARED((NP, DP), jnp.float32),
            pltpu.VMEM_SHARED((NP, DP), jnp.float32),
            pltpu.SemaphoreType.DMA,
            pltpu.SemaphoreType.DMA,
            pltpu.SemaphoreType.DMA((NBUF,)),
            pltpu.SemaphoreType.DMA((NBUF,)),
        ],
    )


def _tc_layer_body(r_ref, p_ref, h_ref, b_ref, w_ref, fin_ref,
                   rout_ref, fout_ref):
    r = r_ref[...]
    v1 = r + p_ref[0] + p_ref[1]
    z = jnp.dot(v1, h_ref[...], preferred_element_type=jnp.float32,
                precision=lax.Precision.HIGHEST) + b_ref[...]
    col = lax.broadcasted_iota(jnp.int32, (NP, DP), 1) < M
    row = lax.broadcasted_iota(jnp.int32, (NP, DP), 0) < N
    rn = jnp.where(col & row, jax.nn.sigmoid(z), 0.0)
    rout_ref[...] = rn
    sgm = rn * w_ref[...]
    mx = jnp.max(jnp.where(col, sgm, -jnp.inf), axis=1, keepdims=True)
    e = jnp.where(col, jnp.exp(sgm - mx), 0.0)
    fl = e / jnp.sum(e, axis=1, keepdims=True)
    fout_ref[...] = fin_ref[...] + jnp.sum(
        jnp.where(row, fl, 0.0), axis=0, keepdims=True)


_tc_layer = pl.pallas_call(
    _tc_layer_body,
    out_shape=[
        jax.ShapeDtypeStruct((NP, DP), jnp.float32),
        jax.ShapeDtypeStruct((1, DP), jnp.float32),
    ],
)


def _final_body(f_ref, xg_ref, wg_ref, bg_ref, wm_ref, bm_ref, out_ref):
    g = jnp.dot(xg_ref[...], wg_ref[...], preferred_element_type=jnp.float32,
                precision=lax.Precision.HIGHEST) + bg_ref[...]
    merged = jnp.concatenate([f_ref[...], g], axis=1)
    out_ref[...] = jnp.dot(merged, wm_ref[...],
                           preferred_element_type=jnp.float32,
                           precision=lax.Precision.HIGHEST) + bm_ref[...]


_final = pl.pallas_call(
    _final_body,
    out_shape=jax.ShapeDtypeStruct((1, 3), jnp.float32),
)


def kernel(x_member, edge_index, x_group, H0, Hs, bH, Ws, Wg, bg, Wm, bm):
    f32 = jnp.float32
    src = edge_index[0].astype(jnp.int32)
    dst = edge_index[1].astype(jnp.int32)
    pad_idx = jnp.full((EP - E,), N, jnp.int32)
    src = jnp.concatenate([src, pad_idx]).reshape(NTILES, K, CH)
    dst = jnp.concatenate([dst, pad_idx]).reshape(NTILES, K, CH)

    r = jnp.pad(x_member.astype(f32), ((0, NP - N), (0, DP - T)))
    zeros_in = jnp.zeros((NP, DP), f32)

    h_pads = [jnp.pad(H0.astype(f32), ((0, DP - T), (0, DP - M)))] + [
        jnp.pad(Hs[i].astype(f32), ((0, DP - M), (0, DP - M)))
        for i in range(R)
    ]
    b_pads = [jnp.pad(bH[i].astype(f32), (0, DP - M)).reshape(1, DP)
              for i in range(R + 1)]
    w_bcast = [jnp.full((1, DP), Ws[i], f32) for i in range(R + 1)]

    f = jnp.zeros((1, DP), f32)
    for layer in range(R + 1):
        partials = _segment_sum_sc()(r, src, dst, zeros_in)
        r, f = _tc_layer(r, partials, h_pads[layer], b_pads[layer],
                         w_bcast[layer], f)

    xg = jnp.pad(x_group.astype(f32), ((0, 0), (0, DP - GLI)))
    wg = jnp.pad(Wg.astype(f32), ((0, DP - GLI), (0, 0)))
    wm = jnp.concatenate(
        [Wm[:M].astype(f32), jnp.zeros((DP - M, 3), f32), Wm[M:].astype(f32)],
        axis=0)
    return _final(f, xg, wg, bg.reshape(1, GLO).astype(f32), wm,
                  bm.reshape(1, 3).astype(f32))


# speedup vs baseline: 1.0161x; 1.0161x over previous
import functools

import jax
import jax.numpy as jnp
from jax import lax
from jax.experimental import pallas as pl
from jax.experimental.pallas import tpu as pltpu
from jax.experimental.pallas import tpu_sc as plsc

N = 10000
T = 6
M = 10
R = 3
E = 640000
GLI = 14
GLO = 16

DP = 16
DS = 10
NTILES = 32
CH = 128
K = 160
NBUF = 8
EP = NTILES * K * CH
ROWS_PER_TILE = 632
NP = 16 * ROWS_PER_TILE

def _segment_sum_body(r_hbm, src_hbm, dst_hbm, zeros_hbm, out_hbm,
                      src_v, dst_v, rows_v, accum_sh, r_sh,
                      zsem, rsem, gsem, ssem):
    c = lax.axis_index("c")
    s = lax.axis_index("s")
    wid = s * 2 + c
    band = pl.ds(s * ROWS_PER_TILE, ROWS_PER_TILE)

    r_cp = pltpu.async_copy(r_hbm.at[band], r_sh.at[band], rsem)
    zero_cp = pltpu.async_copy(zeros_hbm.at[band], accum_sh.at[band], zsem)
    pltpu.sync_copy(src_hbm.at[wid], src_v)
    pltpu.sync_copy(dst_hbm.at[wid], dst_v)
    r_cp.wait()
    zero_cp.wait()
    plsc.subcore_barrier()

    for b in range(NBUF):
        pltpu.async_copy(r_sh.at[src_v.at[b]], rows_v.at[b], gsem.at[b])

    def group(g, carry):
        for b in range(NBUF):
            j = g * NBUF + b
            pltpu.make_async_copy(r_sh.at[src_v.at[j]], rows_v.at[b],
                                  gsem.at[b]).wait()
            pltpu.async_copy(rows_v.at[b], accum_sh.at[dst_v.at[j]],
                             ssem.at[b], add=True).wait()

            @pl.when(j + NBUF < K)
            def _():
                pltpu.async_copy(r_sh.at[src_v.at[j + NBUF]], rows_v.at[b],
                                 gsem.at[b])
        return carry

    lax.fori_loop(0, K // NBUF, group, 0)

    plsc.subcore_barrier()
    pltpu.sync_copy(accum_sh.at[band], out_hbm.at[c, band])


@functools.cache
def _segment_sum_sc():
    mesh = plsc.VectorSubcoreMesh(core_axis_name="c", subcore_axis_name="s")
    return pl.kernel(
        _segment_sum_body,
        out_type=jax.ShapeDtypeStruct((2, NP, DS), jnp.float32),
        mesh=mesh,
        compiler_params=pltpu.CompilerParams(use_tc_tiling_on_sc=False),
        scratch_types=[
            pltpu.VMEM((K, CH), jnp.int32),
            pltpu.VMEM((K, CH), jnp.int32),
            pltpu.VMEM((NBUF, CH, DS), jnp.float32),
            pltpu.VMEM_SHARED((NP, DS), jnp.float32),
            pltpu.VMEM_SHARED((NP, DS), jnp.float32),
            pltpu.SemaphoreType.DMA,
            pltpu.SemaphoreType.DMA,
            pltpu.SemaphoreType.DMA((NBUF,)),
            pltpu.SemaphoreType.DMA((NBUF,)),
        ],
    )


def _tc_layer_body(r_ref, p_ref, h_ref, b_ref, w_ref, fin_ref,
                   rout_ref, fout_ref):
    r = r_ref[...]
    v1 = r + p_ref[0] + p_ref[1]
    z = jnp.dot(v1, h_ref[...], preferred_element_type=jnp.float32,
                precision=lax.Precision.HIGHEST) + b_ref[...]
    col = lax.broadcasted_iota(jnp.int32, (NP, DP), 1) < M
    row = lax.broadcasted_iota(jnp.int32, (NP, DP), 0) < N
    rn = jnp.where(col & row, jax.nn.sigmoid(z), 0.0)
    rout_ref[...] = rn[:, :DS]
    sgm = rn * w_ref[...]
    mx = jnp.max(jnp.where(col, sgm, -jnp.inf), axis=1, keepdims=True)
    e = jnp.where(col, jnp.exp(sgm - mx), 0.0)
    fl = e / jnp.sum(e, axis=1, keepdims=True)
    fout_ref[...] = fin_ref[...] + jnp.sum(
        jnp.where(row, fl, 0.0), axis=0, keepdims=True)


_tc_layer = pl.pallas_call(
    _tc_layer_body,
    out_shape=[
        jax.ShapeDtypeStruct((NP, DS), jnp.float32),
        jax.ShapeDtypeStruct((1, DP), jnp.float32),
    ],
)


def _final_body(f_ref, xg_ref, wg_ref, bg_ref, wm_ref, bm_ref, out_ref):
    g = jnp.dot(xg_ref[...], wg_ref[...], preferred_element_type=jnp.float32,
                precision=lax.Precision.HIGHEST) + bg_ref[...]
    merged = jnp.concatenate([f_ref[...], g], axis=1)
    out_ref[...] = jnp.dot(merged, wm_ref[...],
                           preferred_element_type=jnp.float32,
                           precision=lax.Precision.HIGHEST) + bm_ref[...]


_final = pl.pallas_call(
    _final_body,
    out_shape=jax.ShapeDtypeStruct((1, 3), jnp.float32),
)


def kernel(x_member, edge_index, x_group, H0, Hs, bH, Ws, Wg, bg, Wm, bm):
    f32 = jnp.float32
    src = edge_index[0].astype(jnp.int32)
    dst = edge_index[1].astype(jnp.int32)
    pad_idx = jnp.full((EP - E,), N, jnp.int32)
    src = jnp.concatenate([src, pad_idx]).reshape(NTILES, K, CH)
    dst = jnp.concatenate([dst, pad_idx]).reshape(NTILES, K, CH)

    r = jnp.pad(x_member.astype(f32), ((0, NP - N), (0, DS - T)))
    zeros_in = jnp.zeros((NP, DS), f32)

    h_pads = [jnp.pad(H0.astype(f32), ((0, DS - T), (0, DP - M)))] + [
        jnp.pad(Hs[i].astype(f32), ((0, 0), (0, DP - M)))
        for i in range(R)
    ]
    b_pads = [jnp.pad(bH[i].astype(f32), (0, DP - M)).reshape(1, DP)
              for i in range(R + 1)]
    w_bcast = [jnp.full((1, DP), Ws[i], f32) for i in range(R + 1)]

    f = jnp.zeros((1, DP), f32)
    for layer in range(R + 1):
        partials = _segment_sum_sc()(r, src, dst, zeros_in)
        r, f = _tc_layer(r, partials, h_pads[layer], b_pads[layer],
                         w_bcast[layer], f)

    xg = jnp.pad(x_group.astype(f32), ((0, 0), (0, DP - GLI)))
    wg = jnp.pad(Wg.astype(f32), ((0, DP - GLI), (0, 0)))
    wm = jnp.concatenate(
        [Wm[:M].astype(f32), jnp.zeros((DP - M, 3), f32), Wm[M:].astype(f32)],
        axis=0)
    return _final(f, xg, wg, bg.reshape(1, GLO).astype(f32), wm,
                  bm.reshape(1, 3).astype(f32))


# speedup vs baseline: 1.0177x; 1.0015x over previous
import functools

import jax
import jax.numpy as jnp
from jax import lax
from jax.experimental import pallas as pl
from jax.experimental.pallas import tpu as pltpu
from jax.experimental.pallas import tpu_sc as plsc

N = 10000
T = 6
M = 10
R = 3
E = 640000
GLI = 14
GLO = 16

DP = 16
DS = 16
NTILES = 32
CH = 128
K = 160
NBUF = 8
EP = NTILES * K * CH
ROWS_PER_TILE = 632
NP = 16 * ROWS_PER_TILE

def _segment_sum_body(r_hbm, src_hbm, dst_hbm, zeros_hbm, out_hbm,
                      src_v, dst_v, rows_v, accum_sh, r_sh,
                      zsem, rsem, gsem, ssem):
    c = lax.axis_index("c")
    s = lax.axis_index("s")
    wid = s * 2 + c
    band = pl.ds(s * ROWS_PER_TILE, ROWS_PER_TILE)

    r_cp = pltpu.async_copy(r_hbm.at[band], r_sh.at[band], rsem)
    zero_cp = pltpu.async_copy(zeros_hbm.at[band], accum_sh.at[band], zsem)
    pltpu.sync_copy(src_hbm.at[wid], src_v)
    pltpu.sync_copy(dst_hbm.at[wid], dst_v)
    r_cp.wait()
    zero_cp.wait()
    plsc.subcore_barrier()

    for b in range(NBUF):
        pltpu.async_copy(r_sh.at[src_v.at[b]], rows_v.at[b], gsem.at[b])

    def group(g, carry):
        for b in range(NBUF):
            j = g * NBUF + b
            pltpu.make_async_copy(r_sh.at[src_v.at[j]], rows_v.at[b],
                                  gsem.at[b]).wait()
            pltpu.async_copy(rows_v.at[b], accum_sh.at[dst_v.at[j]],
                             ssem.at[b], add=True).wait()

            @pl.when(j + NBUF < K)
            def _():
                pltpu.async_copy(r_sh.at[src_v.at[j + NBUF]], rows_v.at[b],
                                 gsem.at[b])
        return carry

    lax.fori_loop(0, K // NBUF, group, 0)

    plsc.subcore_barrier()
    pltpu.sync_copy(accum_sh.at[band], out_hbm.at[c, band])


@functools.cache
def _segment_sum_sc():
    mesh = plsc.VectorSubcoreMesh(core_axis_name="c", subcore_axis_name="s")
    return pl.kernel(
        _segment_sum_body,
        out_type=jax.ShapeDtypeStruct((2, NP, DS), jnp.float32),
        mesh=mesh,
        compiler_params=pltpu.CompilerParams(use_tc_tiling_on_sc=False),
        scratch_types=[
            pltpu.VMEM((K, CH), jnp.int32),
            pltpu.VMEM((K, CH), jnp.int32),
            pltpu.VMEM((NBUF, CH, DS), jnp.float32),
            pltpu.VMEM_SHARED((NP, DS), jnp.float32),
            pltpu.VMEM_SHARED((NP, DS), jnp.float32),
            pltpu.SemaphoreType.DMA,
            pltpu.SemaphoreType.DMA,
            pltpu.SemaphoreType.DMA((NBUF,)),
            pltpu.SemaphoreType.DMA((NBUF,)),
        ],
    )


def _tc_layer_body(r_ref, p_ref, h_ref, b_ref, w_ref, fin_ref,
                   rout_ref, fout_ref):
    r = r_ref[...]
    v1 = r + p_ref[0] + p_ref[1]
    z = jnp.dot(v1, h_ref[...], preferred_element_type=jnp.float32,
                precision=lax.Precision.HIGHEST) + b_ref[...]
    col = lax.broadcasted_iota(jnp.int32, (NP, DP), 1) < M
    row = lax.broadcasted_iota(jnp.int32, (NP, DP), 0) < N
    rn = jnp.where(col & row, jax.nn.sigmoid(z), 0.0)
    rout_ref[...] = rn[:, :DS]
    sgm = rn * w_ref[...]
    mx = jnp.max(jnp.where(col, sgm, -jnp.inf), axis=1, keepdims=True)
    e = jnp.where(col, jnp.exp(sgm - mx), 0.0)
    fl = e / jnp.sum(e, axis=1, keepdims=True)
    fout_ref[...] = fin_ref[...] + jnp.sum(
        jnp.where(row, fl, 0.0), axis=0, keepdims=True)


_tc_layer = pl.pallas_call(
    _tc_layer_body,
    out_shape=[
        jax.ShapeDtypeStruct((NP, DS), jnp.float32),
        jax.ShapeDtypeStruct((1, DP), jnp.float32),
    ],
)


def _tc_last_body(r_ref, p_ref, h_ref, b_ref, w_ref, fin_ref,
                  xg_ref, wg_ref, bg_ref, wm_ref, bm_ref, out_ref):
    v1 = r_ref[...] + p_ref[0] + p_ref[1]
    z = jnp.dot(v1, h_ref[...], preferred_element_type=jnp.float32,
                precision=lax.Precision.HIGHEST) + b_ref[...]
    col = lax.broadcasted_iota(jnp.int32, (NP, DP), 1) < M
    row = lax.broadcasted_iota(jnp.int32, (NP, DP), 0) < N
    rn = jnp.where(col & row, jax.nn.sigmoid(z), 0.0)
    sgm = rn * w_ref[...]
    mx = jnp.max(jnp.where(col, sgm, -jnp.inf), axis=1, keepdims=True)
    e = jnp.where(col, jnp.exp(sgm - mx), 0.0)
    fl = e / jnp.sum(e, axis=1, keepdims=True)
    f = fin_ref[...] + jnp.sum(jnp.where(row, fl, 0.0), axis=0, keepdims=True)
    g = jnp.dot(xg_ref[...], wg_ref[...], preferred_element_type=jnp.float32,
                precision=lax.Precision.HIGHEST) + bg_ref[...]
    merged = jnp.concatenate([f, g], axis=1)
    out_ref[...] = jnp.dot(merged, wm_ref[...],
                           preferred_element_type=jnp.float32,
                           precision=lax.Precision.HIGHEST) + bm_ref[...]


_tc_last = pl.pallas_call(
    _tc_last_body,
    out_shape=jax.ShapeDtypeStruct((1, 3), jnp.float32),
)


def kernel(x_member, edge_index, x_group, H0, Hs, bH, Ws, Wg, bg, Wm, bm):
    f32 = jnp.float32
    src = edge_index[0].astype(jnp.int32)
    dst = edge_index[1].astype(jnp.int32)
    pad_idx = jnp.full((EP - E,), N, jnp.int32)
    src = jnp.concatenate([src, pad_idx]).reshape(NTILES, K, CH)
    dst = jnp.concatenate([dst, pad_idx]).reshape(NTILES, K, CH)

    r = jnp.pad(x_member.astype(f32), ((0, NP - N), (0, DS - T)))
    zeros_in = jnp.zeros((NP, DS), f32)

    h_pads = [jnp.pad(H0.astype(f32), ((0, DS - T), (0, DP - M)))] + [
        jnp.pad(Hs[i].astype(f32), ((0, DS - M), (0, DP - M)))
        for i in range(R)
    ]
    b_pads = [jnp.pad(bH[i].astype(f32), (0, DP - M)).reshape(1, DP)
              for i in range(R + 1)]
    w_bcast = [jnp.full((1, DP), Ws[i], f32) for i in range(R + 1)]

    xg = jnp.pad(x_group.astype(f32), ((0, 0), (0, DP - GLI)))
    wg = jnp.pad(Wg.astype(f32), ((0, DP - GLI), (0, 0)))
    wm = jnp.concatenate(
        [Wm[:M].astype(f32), jnp.zeros((DP - M, 3), f32), Wm[M:].astype(f32)],
        axis=0)

    f = jnp.zeros((1, DP), f32)
    for layer in range(R):
        partials = _segment_sum_sc()(r, src, dst, zeros_in)
        r, f = _tc_layer(r, partials, h_pads[layer], b_pads[layer],
                         w_bcast[layer], f)
    partials = _segment_sum_sc()(r, src, dst, zeros_in)
    return _tc_last(r, partials, h_pads[R], b_pads[R], w_bcast[R], f,
                    xg, wg, bg.reshape(1, GLO).astype(f32), wm,
                    bm.reshape(1, 3).astype(f32))
